# baseline (speedup 1.0000x reference)
import functools

import jax
import jax.numpy as jnp
from jax import lax
from jax.experimental import pallas as pl
from jax.experimental.pallas import tpu as pltpu
from jax.experimental.pallas import tpu_sc as plsc

N = 10000
E = 320000
D = 128
L = 3
OUT = 1
EPS = 1e-5

NC = 2
NS = 16
NW = NC * NS
CHUNK = 128
N_PAD = 10112
ROWS_PER_TILE = N_PAD // NS
CHUNKS_PER_W = 80
E_PAD = NW * CHUNK * CHUNKS_PER_W
EPW = E_PAD // NW
NBUF = 2
NI = 4
IPF = 4
UNROLL = 4
CHT = 2 * CHUNKS_PER_W
CH0 = 80
CH1 = CHT - CH0

R = 1000
GRID = N // R


def _sc_agg_body(hn, idx4, zrows, parts, idxr, rows, semI, semG, acc):
    c = lax.axis_index("c")
    s = lax.axis_index("s")
    wid = s * NC + c
    row0 = s * ROWS_PER_TILE

    with jax.named_scope("agg_zero"):
        pltpu.sync_copy(zrows, acc.at[pl.ds(row0, ROWS_PER_TILE)])
        plsc.subcore_barrier()

    base_t = s * CHT + c * CH0
    NB = jnp.where(c == 0, CH0 // UNROLL, CH1 // UNROLL)

    def idx_load(k, ji):
        pltpu.async_copy(idx4.at[base_t + k], idxr.at[ji], semI.at[ji])

    def idx_wait(k, ji):
        pltpu.make_async_copy(idx4.at[base_t + k], idxr.at[ji], semI.at[ji]).wait()

    with jax.named_scope("agg_prime"):
        for m in range(IPF):
            idx_load(m, m % NI)
        for m in range(NBUF):
            idx_wait(m, m % NI)
            pltpu.async_copy(hn.at[idxr.at[m % NI, 0]], rows.at[m % NBUF],
                             semG.at[m % NBUF])

    scope_edges = jax.named_scope("agg_edges")
    scope_edges.__enter__()

    @pl.loop(0, NB)
    def _loop_body(b):
        k0 = b * UNROLL
        for i in range(UNROLL):
            k = k0 + i
            j = i % NBUF
            ji = i % NI
            jg = (i + NBUF) % NI
            pltpu.make_async_copy(hn.at[idxr.at[ji, 0]], rows.at[j],
                                  semG.at[j]).wait()
            pltpu.sync_copy(rows.at[j], acc.at[idxr.at[ji, 1]], add=True)
            def _refill(jn=j, jg=jg):
                pltpu.async_copy(hn.at[idxr.at[jg, 0]], rows.at[jn],
                                 semG.at[jn])
            if i < UNROLL - NBUF:
                idx_wait(k + NBUF, jg)
                _refill()
            else:
                @pl.when(b < NB - 1)
                def _():
                    idx_wait(k + NBUF, jg)
                    _refill()
            @pl.when(b < NB - 1)
            def _():
                idx_load(k + IPF, ji)

    scope_edges.__exit__(None, None, None)

    with jax.named_scope("agg_wb"):
        plsc.subcore_barrier()
        pltpu.sync_copy(acc.at[pl.ds(row0, ROWS_PER_TILE)],
                        parts.at[c, pl.ds(row0, ROWS_PER_TILE)])


_DEG_K = 8


def _sc_deg_body(dst3, zrows, ones, degparts, didx, onesv, sem, acc):
    c = lax.axis_index("c")
    s = lax.axis_index("s")
    wid = s * NC + c
    row0 = s * ROWS_PER_TILE

    pltpu.sync_copy(zrows, acc.at[pl.ds(row0, ROWS_PER_TILE)])
    pltpu.sync_copy(ones, onesv)
    pltpu.sync_copy(dst3.at[wid], didx)
    plsc.subcore_barrier()

    @pl.loop(0, CHUNKS_PER_W // _DEG_K)
    def _(b):
        k0 = b * _DEG_K
        for j in range(_DEG_K):
            pltpu.async_copy(onesv, acc.at[didx.at[k0 + j]], sem, add=True)
        for j in range(_DEG_K):
            pltpu.make_async_copy(onesv, acc.at[didx.at[k0 + j]], sem).wait()

    plsc.subcore_barrier()
    pltpu.sync_copy(acc.at[pl.ds(row0, ROWS_PER_TILE)],
                    degparts.at[c, pl.ds(row0, ROWS_PER_TILE)])


def _mesh():
    return plsc.VectorSubcoreMesh(core_axis_name="c", subcore_axis_name="s",
                                  num_cores=NC, num_subcores=NS)


@functools.cache
def _make_sc_agg():
    return pl.kernel(
        _sc_agg_body,
        out_type=jax.ShapeDtypeStruct((NC, N_PAD, D), jnp.float32),
        mesh=_mesh(),
        scratch_types=(
            pltpu.VMEM((NI, 2, CHUNK), jnp.int32),
            pltpu.VMEM((NBUF, CHUNK, D), jnp.float32),
            pltpu.SemaphoreType.DMA((NI,)),
            pltpu.SemaphoreType.DMA((NBUF,)),
            pltpu.VMEM_SHARED((N_PAD, D), jnp.float32),
        ),
    )


@functools.cache
def _make_sc_deg():
    return pl.kernel(
        _sc_deg_body,
        out_type=jax.ShapeDtypeStruct((NC, N_PAD, D), jnp.float32),
        mesh=_mesh(),
        scratch_types=(
            pltpu.VMEM((CHUNKS_PER_W, CHUNK), jnp.int32),
            pltpu.VMEM((CHUNK, D), jnp.float32),
            pltpu.SemaphoreType.DMA,
            pltpu.VMEM_SHARED((N_PAD, D), jnp.float32),
        ),
    )


def _ln(h, scale, bias):
    mu = jnp.mean(h, axis=1, keepdims=True)
    d = h - mu
    var = jnp.mean(d * d, axis=1, keepdims=True)
    return d * lax.rsqrt(var + EPS) * scale + bias


def _tc_in_body(x, wt, b, sc, bn, h_out, hn_out):
    h = jnp.dot(x[...], wt[...], preferred_element_type=jnp.float32) + b[...]
    h_out[...] = h
    hn_out[...] = _ln(h, sc[...], bn[...])


def _tc_layer_body(h, hn, p0, p1, d0, d1, wlt, bl, wrt, sc, bn, h_out, hn_out):
    deg = jnp.maximum(d0[...][:, :1] + d1[...][:, :1], 1.0)
    agg = (p0[...] + p1[...]) / deg
    conv = (jnp.dot(agg, wlt[...], preferred_element_type=jnp.float32) + bl[...]
            + jnp.dot(hn[...], wrt[...], preferred_element_type=jnp.float32))
    hnew = jnp.maximum(h[...] + conv, 0.0)
    h_out[...] = hnew
    hn_out[...] = _ln(hnew, sc[...], bn[...])


def _tc_last_body(h, hn, p0, p1, d0, d1, wlt, bl, wrt, owt, ob, y_out):
    deg = jnp.maximum(d0[...][:, :1] + d1[...][:, :1], 1.0)
    agg = (p0[...] + p1[...]) / deg
    conv = (jnp.dot(agg, wlt[...], preferred_element_type=jnp.float32) + bl[...]
            + jnp.dot(hn[...], wrt[...], preferred_element_type=jnp.float32))
    hnew = jnp.maximum(h[...] + conv, 0.0)
    y_out[...] = jnp.dot(hnew, owt[...], preferred_element_type=jnp.float32) + ob[...]


def _row_spec(width=D):
    return pl.BlockSpec((R, width), lambda i: (i, 0))


def _full_spec(shape):
    return pl.BlockSpec(shape, lambda i: tuple(0 for _ in shape))


_tc_in = pl.pallas_call(
    _tc_in_body,
    grid=(GRID,),
    in_specs=[_row_spec(), _full_spec((D, D)), _full_spec((1, D)),
              _full_spec((1, D)), _full_spec((1, D))],
    out_specs=[_row_spec(), _row_spec()],
    out_shape=[jax.ShapeDtypeStruct((N, D), jnp.float32),
               jax.ShapeDtypeStruct((N, D), jnp.float32)],
)

_tc_layer = pl.pallas_call(
    _tc_layer_body,
    grid=(GRID,),
    in_specs=[_row_spec(), _row_spec(), _row_spec(), _row_spec(),
              _row_spec(16), _row_spec(16),
              _full_spec((D, D)), _full_spec((1, D)), _full_spec((D, D)),
              _full_spec((1, D)), _full_spec((1, D))],
    out_specs=[_row_spec(), _row_spec()],
    out_shape=[jax.ShapeDtypeStruct((N, D), jnp.float32),
               jax.ShapeDtypeStruct((N, D), jnp.float32)],
)

_tc_last = pl.pallas_call(
    _tc_last_body,
    grid=(GRID,),
    in_specs=[_row_spec(), _row_spec(), _row_spec(), _row_spec(),
              _row_spec(16), _row_spec(16),
              _full_spec((D, D)), _full_spec((1, D)), _full_spec((D, D)),
              _full_spec((D, OUT)), _full_spec((1, OUT))],
    out_specs=[_row_spec(OUT)],
    out_shape=[jax.ShapeDtypeStruct((N, OUT), jnp.float32)],
)


def kernel(x, edge_index, in_W, in_b, lin_l_W, lin_l_b, lin_r_W,
           ln_scale, ln_bias, out_W, out_b):
    src = edge_index[0]
    dst = edge_index[1]
    pad = E_PAD - E
    srcp = jnp.concatenate([src, jnp.zeros((pad,), jnp.int32)])
    pad_dst = N + (jnp.arange(pad, dtype=jnp.int32) % (N_PAD - N))
    dstp = jnp.concatenate([dst, pad_dst])
    src3 = srcp.reshape(NW, CHUNKS_PER_W, CHUNK)
    dst3 = dstp.reshape(NW, CHUNKS_PER_W, CHUNK)
    idx4 = jnp.stack([srcp.reshape(-1, CHUNK), dstp.reshape(-1, CHUNK)],
                     axis=1)
    zrows = jnp.zeros((ROWS_PER_TILE, D), jnp.float32)
    ones = jnp.ones((CHUNK, D), jnp.float32)

    h, hn = _tc_in(x, in_W.T, in_b[None], ln_scale[0][None], ln_bias[0][None])

    degparts = _make_sc_deg()(dst3, zrows, ones)
    parts = _make_sc_agg()(hn, idx4, zrows)
    d0 = degparts[0, :N, :16]
    d1 = degparts[1, :N, :16]

    for i in range(L):
        p0 = parts[0, :N]
        p1 = parts[1, :N]
        if i < L - 1:
            h, hn = _tc_layer(h, hn, p0, p1, d0, d1,
                              lin_l_W[i].T, lin_l_b[i][None], lin_r_W[i].T,
                              ln_scale[i + 1][None], ln_bias[i + 1][None])
            parts = _make_sc_agg()(hn, idx4, zrows)
        else:
            y = _tc_last(h, hn, p0, p1, d0, d1,
                         lin_l_W[i].T, lin_l_b[i][None], lin_r_W[i].T,
                         out_W.T, out_b[None])
    (y,) = y if isinstance(y, (list, tuple)) else (y,)
    return y


# speedup vs baseline: 3.1570x; 3.1570x over previous
import functools

import jax
import jax.numpy as jnp
from jax import lax
from jax.experimental import pallas as pl
from jax.experimental.pallas import tpu as pltpu
from jax.experimental.pallas import tpu_sc as plsc

N = 10000
E = 320000
D = 128
L = 3
OUT = 1
EPS = 1e-5

NC = 2
NS = 16
NW = NC * NS
CHUNK = 128
N_PAD = 10112
ROWS_PER_TILE = N_PAD // NS
CHUNKS_PER_W = 80
E_PAD = NW * CHUNK * CHUNKS_PER_W
EPW = E_PAD // NW
NBUF = 2
NI = 4
IPF = 4
UNROLL = 4
CHT = 2 * CHUNKS_PER_W
CH0 = 80
CH1 = CHT - CH0

R = 1000
GRID = N // R


def _sc_agg_body(hn, idx4, zrows, parts, idxr, rows, semI, semG, acc):
    c = lax.axis_index("c")
    s = lax.axis_index("s")
    wid = s * NC + c
    row0 = s * ROWS_PER_TILE

    with jax.named_scope("agg_zero"):
        pltpu.sync_copy(zrows, acc.at[pl.ds(row0, ROWS_PER_TILE)])
        plsc.subcore_barrier()

    base_t = s * CHT + c * CH0
    NB = jnp.where(c == 0, CH0 // UNROLL, CH1 // UNROLL)

    def idx_load(k, ji):
        pltpu.async_copy(idx4.at[base_t + k], idxr.at[ji], semI.at[ji])

    def idx_wait(k, ji):
        pltpu.make_async_copy(idx4.at[base_t + k], idxr.at[ji], semI.at[ji]).wait()

    with jax.named_scope("agg_prime"):
        for m in range(IPF):
            idx_load(m, m % NI)
        for m in range(NBUF):
            idx_wait(m, m % NI)
            pltpu.async_copy(hn.at[idxr.at[m % NI, 0]], rows.at[m % NBUF],
                             semG.at[m % NBUF])

    scope_edges = jax.named_scope("agg_edges")
    scope_edges.__enter__()

    @pl.loop(0, NB)
    def _loop_body(b):
        k0 = b * UNROLL
        for i in range(UNROLL):
            k = k0 + i
            j = i % NBUF
            ji = i % NI
            jg = (i + NBUF) % NI
            pltpu.make_async_copy(hn.at[idxr.at[ji, 0]], rows.at[j],
                                  semG.at[j]).wait()
            pltpu.sync_copy(rows.at[j], acc.at[idxr.at[ji, 1]], add=True)
            def _refill(jn=j, jg=jg):
                pltpu.async_copy(hn.at[idxr.at[jg, 0]], rows.at[jn],
                                 semG.at[jn])
            if i < UNROLL - NBUF:
                idx_wait(k + NBUF, jg)
                _refill()
            else:
                @pl.when(b < NB - 1)
                def _():
                    idx_wait(k + NBUF, jg)
                    _refill()
            @pl.when(b < NB - 1)
            def _():
                idx_load(k + IPF, ji)

    scope_edges.__exit__(None, None, None)

    with jax.named_scope("agg_wb"):
        plsc.subcore_barrier()
        pltpu.sync_copy(acc.at[pl.ds(row0, ROWS_PER_TILE)],
                        parts.at[c, pl.ds(row0, ROWS_PER_TILE)])


_DEG_K = 8


def _sc_deg_body(dst3, zrows, ones, degparts, didx, onesv, sem, acc):
    c = lax.axis_index("c")
    s = lax.axis_index("s")
    wid = s * NC + c
    row0 = s * ROWS_PER_TILE

    pltpu.sync_copy(zrows, acc.at[pl.ds(row0, ROWS_PER_TILE)])
    pltpu.sync_copy(ones, onesv)
    pltpu.sync_copy(dst3.at[wid], didx)
    plsc.subcore_barrier()

    @pl.loop(0, CHUNKS_PER_W // _DEG_K)
    def _(b):
        k0 = b * _DEG_K
        for j in range(_DEG_K):
            pltpu.async_copy(onesv, acc.at[didx.at[k0 + j]], sem, add=True)
        for j in range(_DEG_K):
            pltpu.make_async_copy(onesv, acc.at[didx.at[k0 + j]], sem).wait()

    plsc.subcore_barrier()
    pltpu.sync_copy(acc.at[pl.ds(row0, ROWS_PER_TILE)],
                    degparts.at[c, pl.ds(row0, ROWS_PER_TILE)])


def _mesh():
    return plsc.VectorSubcoreMesh(core_axis_name="c", subcore_axis_name="s",
                                  num_cores=NC, num_subcores=NS)


@functools.cache
def _make_sc_agg():
    return pl.kernel(
        _sc_agg_body,
        out_type=jax.ShapeDtypeStruct((NC, N_PAD, D), jnp.float32),
        mesh=_mesh(),
        scratch_types=(
            pltpu.VMEM((NI, 2, CHUNK), jnp.int32),
            pltpu.VMEM((NBUF, CHUNK, D), jnp.float32),
            pltpu.SemaphoreType.DMA((NI,)),
            pltpu.SemaphoreType.DMA((NBUF,)),
            pltpu.VMEM_SHARED((N_PAD, D), jnp.float32),
        ),
    )


@functools.cache
def _make_sc_deg():
    return pl.kernel(
        _sc_deg_body,
        out_type=jax.ShapeDtypeStruct((NC, N_PAD, D), jnp.float32),
        mesh=_mesh(),
        scratch_types=(
            pltpu.VMEM((CHUNKS_PER_W, CHUNK), jnp.int32),
            pltpu.VMEM((CHUNK, D), jnp.float32),
            pltpu.SemaphoreType.DMA,
            pltpu.VMEM_SHARED((N_PAD, D), jnp.float32),
        ),
    )


def _ln(h, scale, bias):
    mu = jnp.mean(h, axis=1, keepdims=True)
    d = h - mu
    var = jnp.mean(d * d, axis=1, keepdims=True)
    return d * lax.rsqrt(var + EPS) * scale + bias


def _tc_in_body(x, wt, b, sc, bn, h_out, hn_out):
    h = jnp.dot(x[...], wt[...], preferred_element_type=jnp.float32) + b[...]
    h_out[...] = h
    hn_out[...] = _ln(h, sc[...], bn[...])


def _tc_layer_body(h, hn, p0, p1, d0, d1, wlt, bl, wrt, sc, bn, h_out, hn_out):
    deg = jnp.maximum(d0[...][:, :1] + d1[...][:, :1], 1.0)
    agg = (p0[...] + p1[...]) / deg
    conv = (jnp.dot(agg, wlt[...], preferred_element_type=jnp.float32) + bl[...]
            + jnp.dot(hn[...], wrt[...], preferred_element_type=jnp.float32))
    hnew = jnp.maximum(h[...] + conv, 0.0)
    h_out[...] = hnew
    hn_out[...] = _ln(hnew, sc[...], bn[...])


def _tc_last_body(h, hn, p0, p1, d0, d1, wlt, bl, wrt, owt, ob, y_out):
    deg = jnp.maximum(d0[...][:, :1] + d1[...][:, :1], 1.0)
    agg = (p0[...] + p1[...]) / deg
    conv = (jnp.dot(agg, wlt[...], preferred_element_type=jnp.float32) + bl[...]
            + jnp.dot(hn[...], wrt[...], preferred_element_type=jnp.float32))
    hnew = jnp.maximum(h[...] + conv, 0.0)
    y_out[...] = jnp.dot(hnew, owt[...], preferred_element_type=jnp.float32) + ob[...]


def _row_spec(width=D):
    return pl.BlockSpec((R, width), lambda i: (i, 0))


def _full_spec(shape):
    return pl.BlockSpec(shape, lambda i: tuple(0 for _ in shape))


_tc_in = pl.pallas_call(
    _tc_in_body,
    grid=(GRID,),
    in_specs=[_row_spec(), _full_spec((D, D)), _full_spec((1, D)),
              _full_spec((1, D)), _full_spec((1, D))],
    out_specs=[_row_spec(), _row_spec()],
    out_shape=[jax.ShapeDtypeStruct((N, D), jnp.float32),
               jax.ShapeDtypeStruct((N, D), jnp.float32)],
)

_tc_layer = pl.pallas_call(
    _tc_layer_body,
    grid=(GRID,),
    in_specs=[_row_spec(), _row_spec(), _row_spec(), _row_spec(),
              _row_spec(16), _row_spec(16),
              _full_spec((D, D)), _full_spec((1, D)), _full_spec((D, D)),
              _full_spec((1, D)), _full_spec((1, D))],
    out_specs=[_row_spec(), _row_spec()],
    out_shape=[jax.ShapeDtypeStruct((N, D), jnp.float32),
               jax.ShapeDtypeStruct((N, D), jnp.float32)],
)

_tc_last = pl.pallas_call(
    _tc_last_body,
    grid=(GRID,),
    in_specs=[_row_spec(), _row_spec(), _row_spec(), _row_spec(),
              _row_spec(16), _row_spec(16),
              _full_spec((D, D)), _full_spec((1, D)), _full_spec((D, D)),
              _full_spec((D, OUT)), _full_spec((1, OUT))],
    out_specs=[_row_spec(OUT)],
    out_shape=[jax.ShapeDtypeStruct((N, OUT), jnp.float32)],
)


def kernel(x, edge_index, in_W, in_b, lin_l_W, lin_l_b, lin_r_W,
           ln_scale, ln_bias, out_W, out_b):
    src = edge_index[0]
    dst = edge_index[1]
    pad = E_PAD - E
    pad_src = (jnp.arange(pad, dtype=jnp.int32) * 79) % N
    srcp = jnp.concatenate([src, pad_src])
    pad_dst = N + (jnp.arange(pad, dtype=jnp.int32) % (N_PAD - N))
    dstp = jnp.concatenate([dst, pad_dst])
    src3 = srcp.reshape(NW, CHUNKS_PER_W, CHUNK)
    dst3 = dstp.reshape(NW, CHUNKS_PER_W, CHUNK)
    idx4 = jnp.stack([srcp.reshape(-1, CHUNK), dstp.reshape(-1, CHUNK)],
                     axis=1)
    zrows = jnp.zeros((ROWS_PER_TILE, D), jnp.float32)
    ones = jnp.ones((CHUNK, D), jnp.float32)

    h, hn = _tc_in(x, in_W.T, in_b[None], ln_scale[0][None], ln_bias[0][None])

    degparts = _make_sc_deg()(dst3, zrows, ones)
    parts = _make_sc_agg()(hn, idx4, zrows)
    d0 = degparts[0, :N, :16]
    d1 = degparts[1, :N, :16]

    for i in range(L):
        p0 = parts[0, :N]
        p1 = parts[1, :N]
        if i < L - 1:
            h, hn = _tc_layer(h, hn, p0, p1, d0, d1,
                              lin_l_W[i].T, lin_l_b[i][None], lin_r_W[i].T,
                              ln_scale[i + 1][None], ln_bias[i + 1][None])
            parts = _make_sc_agg()(hn, idx4, zrows)
        else:
            y = _tc_last(h, hn, p0, p1, d0, d1,
                         lin_l_W[i].T, lin_l_b[i][None], lin_r_W[i].T,
                         out_W.T, out_b[None])
    (y,) = y if isinstance(y, (list, tuple)) else (y,)
    return y


# speedup vs baseline: 3.3293x; 1.0546x over previous
import functools

import jax
import jax.numpy as jnp
from jax import lax
from jax.experimental import pallas as pl
from jax.experimental.pallas import tpu as pltpu
from jax.experimental.pallas import tpu_sc as plsc

N = 10000
E = 320000
D = 128
L = 3
OUT = 1
EPS = 1e-5

NC = 2
NS = 16
NW = NC * NS
CHUNK = 128
N_PAD = 10112
ROWS_PER_TILE = N_PAD // NS
CHUNKS_PER_W = 80
E_PAD = NW * CHUNK * CHUNKS_PER_W
EPW = E_PAD // NW
NBUF = 2
NI = 4
IPF = 4
UNROLL = 4
CHT = 2 * CHUNKS_PER_W
CH0 = 80
CH1 = CHT - CH0

R = 1000
GRID = N // R


def _sc_agg_body(hn, idx4, zrows, parts, idxr, rows, semI, semG, acc):
    c = lax.axis_index("c")
    s = lax.axis_index("s")
    wid = s * NC + c
    row0 = s * ROWS_PER_TILE

    with jax.named_scope("agg_zero"):
        pltpu.sync_copy(zrows, acc.at[pl.ds(row0, ROWS_PER_TILE)])
        plsc.subcore_barrier()

    base_t = s * CHT + c * CH0
    NB = jnp.where(c == 0, CH0 // UNROLL, CH1 // UNROLL)

    def idx_load(k, ji):
        pltpu.async_copy(idx4.at[base_t + k], idxr.at[ji], semI.at[ji])

    def idx_wait(k, ji):
        pltpu.make_async_copy(idx4.at[base_t + k], idxr.at[ji], semI.at[ji]).wait()

    with jax.named_scope("agg_prime"):
        for m in range(IPF):
            idx_load(m, m % NI)
        for m in range(NBUF):
            idx_wait(m, m % NI)
            pltpu.async_copy(hn.at[idxr.at[m % NI, 0]], rows.at[m % NBUF],
                             semG.at[m % NBUF])

    scope_edges = jax.named_scope("agg_edges")
    scope_edges.__enter__()

    @pl.loop(0, NB)
    def _loop_body(b):
        k0 = b * UNROLL
        for i in range(UNROLL):
            k = k0 + i
            j = i % NBUF
            ji = i % NI
            jg = (i + NBUF) % NI
            pltpu.make_async_copy(hn.at[idxr.at[ji, 0]], rows.at[j],
                                  semG.at[j]).wait()
            pltpu.sync_copy(rows.at[j], acc.at[idxr.at[ji, 1]], add=True)
            def _refill(jn=j, jg=jg):
                pltpu.async_copy(hn.at[idxr.at[jg, 0]], rows.at[jn],
                                 semG.at[jn])
            if i < UNROLL - NBUF:
                idx_wait(k + NBUF, jg)
                _refill()
            else:
                @pl.when(b < NB - 1)
                def _():
                    idx_wait(k + NBUF, jg)
                    _refill()
            @pl.when(b < NB - 1)
            def _():
                idx_load(k + IPF, ji)

    scope_edges.__exit__(None, None, None)

    with jax.named_scope("agg_wb"):
        plsc.subcore_barrier()
        pltpu.sync_copy(acc.at[pl.ds(row0, ROWS_PER_TILE)],
                        parts.at[c, pl.ds(row0, ROWS_PER_TILE)])


_DEG_K = 8


def _sc_deg_body(dst3, zrows, ones, degparts, didx, onesv, sem, acc):
    c = lax.axis_index("c")
    s = lax.axis_index("s")
    wid = s * NC + c
    row0 = s * ROWS_PER_TILE

    pltpu.sync_copy(zrows, acc.at[pl.ds(row0, ROWS_PER_TILE)])
    pltpu.sync_copy(ones, onesv)
    pltpu.sync_copy(dst3.at[wid], didx)
    plsc.subcore_barrier()

    @pl.loop(0, CHUNKS_PER_W // _DEG_K)
    def _(b):
        k0 = b * _DEG_K
        for j in range(_DEG_K):
            pltpu.async_copy(onesv, acc.at[didx.at[k0 + j]], sem, add=True)
        for j in range(_DEG_K):
            pltpu.make_async_copy(onesv, acc.at[didx.at[k0 + j]], sem).wait()

    plsc.subcore_barrier()
    pltpu.sync_copy(acc.at[pl.ds(row0, ROWS_PER_TILE)],
                    degparts.at[c, pl.ds(row0, ROWS_PER_TILE)])


def _mesh():
    return plsc.VectorSubcoreMesh(core_axis_name="c", subcore_axis_name="s",
                                  num_cores=NC, num_subcores=NS)


@functools.cache
def _make_sc_agg():
    return pl.kernel(
        _sc_agg_body,
        out_type=jax.ShapeDtypeStruct((NC, N_PAD, D), jnp.float32),
        mesh=_mesh(),
        scratch_types=(
            pltpu.VMEM((NI, 2, CHUNK), jnp.int32),
            pltpu.VMEM((NBUF, CHUNK, D), jnp.float32),
            pltpu.SemaphoreType.DMA((NI,)),
            pltpu.SemaphoreType.DMA((NBUF,)),
            pltpu.VMEM_SHARED((N_PAD, D), jnp.float32),
        ),
    )


@functools.cache
def _make_sc_deg():
    return pl.kernel(
        _sc_deg_body,
        out_type=jax.ShapeDtypeStruct((NC, N_PAD, D), jnp.float32),
        mesh=_mesh(),
        scratch_types=(
            pltpu.VMEM((CHUNKS_PER_W, CHUNK), jnp.int32),
            pltpu.VMEM((CHUNK, D), jnp.float32),
            pltpu.SemaphoreType.DMA,
            pltpu.VMEM_SHARED((N_PAD, D), jnp.float32),
        ),
    )


def _ln(h, scale, bias):
    mu = jnp.mean(h, axis=1, keepdims=True)
    d = h - mu
    var = jnp.mean(d * d, axis=1, keepdims=True)
    return d * lax.rsqrt(var + EPS) * scale + bias


def _tc_in_body(x, wt, b, sc, bn, h_out, hn_out):
    h = jnp.dot(x[...], wt[...], preferred_element_type=jnp.float32) + b[...]
    h_out[...] = h
    hn_out[...] = _ln(h, sc[...], bn[...])


def _tc_layer_body(h, hn, pp, dp, wlt, bl, wrt, sc, bn, h_out, hn_out):
    deg = jnp.maximum(dp[0][:, :1] + dp[1][:, :1], 1.0)
    agg = (pp[0] + pp[1]) / deg
    conv = (jnp.dot(agg, wlt[...], preferred_element_type=jnp.float32) + bl[...]
            + jnp.dot(hn[...], wrt[...], preferred_element_type=jnp.float32))
    hnew = jnp.maximum(h[...] + conv, 0.0)
    h_out[...] = hnew
    hn_out[...] = _ln(hnew, sc[...], bn[...])


def _tc_last_body(h, hn, pp, dp, wlt, bl, wrt, owt, ob, y_out):
    deg = jnp.maximum(dp[0][:, :1] + dp[1][:, :1], 1.0)
    agg = (pp[0] + pp[1]) / deg
    conv = (jnp.dot(agg, wlt[...], preferred_element_type=jnp.float32) + bl[...]
            + jnp.dot(hn[...], wrt[...], preferred_element_type=jnp.float32))
    hnew = jnp.maximum(h[...] + conv, 0.0)
    y_out[...] = jnp.dot(hnew, owt[...], preferred_element_type=jnp.float32) + ob[...]


def _row_spec(width=D):
    return pl.BlockSpec((R, width), lambda i: (i, 0))


def _full_spec(shape):
    return pl.BlockSpec(shape, lambda i: tuple(0 for _ in shape))


def _parts_spec():
    return pl.BlockSpec((NC, R, D), lambda i: (0, i, 0))


_tc_in = pl.pallas_call(
    _tc_in_body,
    grid=(GRID,),
    in_specs=[_row_spec(), _full_spec((D, D)), _full_spec((1, D)),
              _full_spec((1, D)), _full_spec((1, D))],
    out_specs=[_row_spec(), _row_spec()],
    out_shape=[jax.ShapeDtypeStruct((N, D), jnp.float32),
               jax.ShapeDtypeStruct((N, D), jnp.float32)],
)

_tc_layer = pl.pallas_call(
    _tc_layer_body,
    grid=(GRID,),
    in_specs=[_row_spec(), _row_spec(), _parts_spec(), _parts_spec(),
              _full_spec((D, D)), _full_spec((1, D)), _full_spec((D, D)),
              _full_spec((1, D)), _full_spec((1, D))],
    out_specs=[_row_spec(), _row_spec()],
    out_shape=[jax.ShapeDtypeStruct((N, D), jnp.float32),
               jax.ShapeDtypeStruct((N, D), jnp.float32)],
)

_tc_last = pl.pallas_call(
    _tc_last_body,
    grid=(GRID,),
    in_specs=[_row_spec(), _row_spec(), _parts_spec(), _parts_spec(),
              _full_spec((D, D)), _full_spec((1, D)), _full_spec((D, D)),
              _full_spec((D, OUT)), _full_spec((1, OUT))],
    out_specs=[_row_spec(OUT)],
    out_shape=[jax.ShapeDtypeStruct((N, OUT), jnp.float32)],
)


def kernel(x, edge_index, in_W, in_b, lin_l_W, lin_l_b, lin_r_W,
           ln_scale, ln_bias, out_W, out_b):
    src = edge_index[0]
    dst = edge_index[1]
    pad = E_PAD - E
    pad_src = (jnp.arange(pad, dtype=jnp.int32) * 79) % N
    srcp = jnp.concatenate([src, pad_src])
    pad_dst = N + (jnp.arange(pad, dtype=jnp.int32) % (N_PAD - N))
    dstp = jnp.concatenate([dst, pad_dst])
    src3 = srcp.reshape(NW, CHUNKS_PER_W, CHUNK)
    dst3 = dstp.reshape(NW, CHUNKS_PER_W, CHUNK)
    idx4 = jnp.stack([srcp.reshape(-1, CHUNK), dstp.reshape(-1, CHUNK)],
                     axis=1)
    zrows = jnp.zeros((ROWS_PER_TILE, D), jnp.float32)
    ones = jnp.ones((CHUNK, D), jnp.float32)

    h, hn = _tc_in(x, in_W.T, in_b[None], ln_scale[0][None], ln_bias[0][None])

    degparts = _make_sc_deg()(dst3, zrows, ones)
    parts = _make_sc_agg()(hn, idx4, zrows)

    for i in range(L):
        if i < L - 1:
            h, hn = _tc_layer(h, hn, parts, degparts,
                              lin_l_W[i].T, lin_l_b[i][None], lin_r_W[i].T,
                              ln_scale[i + 1][None], ln_bias[i + 1][None])
            parts = _make_sc_agg()(hn, idx4, zrows)
        else:
            y = _tc_last(h, hn, parts, degparts,
                         lin_l_W[i].T, lin_l_b[i][None], lin_r_W[i].T,
                         out_W.T, out_b[None])
    (y,) = y if isinstance(y, (list, tuple)) else (y,)
    return y
